# Initial kernel scaffold; baseline (speedup 1.0000x reference)
import jax
import jax.numpy as jnp
from jax.experimental import pallas as pl


def kernel(tokens, segments, token_table, segment_table, pos_weight):
    raise NotImplementedError("write your pallas kernel here")


# speedup vs baseline: 3.5088x; 3.5088x over previous
import functools

import jax
import jax.numpy as jnp
from jax import lax
from jax.experimental import pallas as pl
from jax.experimental.pallas import tpu as pltpu
from jax.experimental.pallas import tpu_sc as plsc

_H = 128
_T = 200
_NW = 32
_ROWS_PER_W = 32
_HV = _H // 16


def _emb_body(tok_hbm, seg_hbm, table_hbm, segtab_hbm, pos_hbm, out_hbm,
              tok_v, seg_v, rows_v, posab_v, segtab_v, sem):
    c = lax.axis_index("c")
    s = lax.axis_index("s")
    wid = s * 2 + c
    base0 = wid * (_ROWS_PER_W * _T)

    pltpu.sync_copy(pos_hbm, rows_v.at[0])
    pltpu.sync_copy(segtab_hbm, segtab_v)

    def pab_body(r, carry):
        for h in range(_HV):
            sl = pl.ds(h * 16, 16)
            p = rows_v[0, r, sl]
            posab_v[r, sl] = p + segtab_v[0, sl]
            posab_v[_T + r, sl] = p + segtab_v[1, sl]
        return carry

    lax.fori_loop(0, _T, pab_body, 0)

    def row_body(j, carry):
        base = base0 + j * _T
        pltpu.sync_copy(tok_hbm.at[pl.ds(base, 128)], tok_v.at[pl.ds(0, 128)])
        pltpu.sync_copy(tok_hbm.at[pl.ds(base + 128, _T - 128)],
                        tok_v.at[pl.ds(128, _T - 128)])
        pltpu.sync_copy(seg_hbm.at[pl.ds(base, _T)], seg_v.at[pl.ds(0, _T)])
        cp1 = pltpu.async_copy(table_hbm.at[tok_v.at[pl.ds(0, 128)]],
                               rows_v.at[0, pl.ds(0, 128)], sem)
        cp2 = pltpu.async_copy(table_hbm.at[tok_v.at[pl.ds(128, _T - 128)]],
                               rows_v.at[0, pl.ds(128, _T - 128)], sem)
        cp1.wait()
        cp2.wait()

        def comp(rv, carry2):
            sv = seg_v[pl.ds(rv * 16, 16)]
            av = sv * _T + (rv * 16 + lax.iota(jnp.int32, 16))
            for u in range(16):
                r = rv * 16 + u
                a = av[u]
                for h in range(_HV):
                    sl = pl.ds(h * 16, 16)
                    rows_v[0, r, sl] = rows_v[0, r, sl] + posab_v[a, sl]
            return carry2

        lax.fori_loop(0, _T // 16, comp, 0)
        sv = seg_v[pl.ds(192, 16)]
        av = sv * _T + (192 + lax.iota(jnp.int32, 16))
        for u in range(_T - 192):
            r = 192 + u
            a = av[u]
            for h in range(_HV):
                sl = pl.ds(h * 16, 16)
                rows_v[0, r, sl] = rows_v[0, r, sl] + posab_v[a, sl]
        pltpu.sync_copy(rows_v.at[0], out_hbm.at[pl.ds(base, _T)])
        return carry

    lax.fori_loop(0, _ROWS_PER_W, row_body, 0)


def _emb(tok, seg, token_table, segment_table, pos_weight):
    n = tok.shape[0]
    mesh = plsc.VectorSubcoreMesh(core_axis_name="c", subcore_axis_name="s")
    return pl.kernel(
        _emb_body,
        out_type=jax.ShapeDtypeStruct((n, _H), jnp.float32),
        mesh=mesh,
        scratch_types=[
            pltpu.VMEM((256,), jnp.int32),
            pltpu.VMEM((256,), jnp.int32),
            pltpu.VMEM((1, _T, _H), jnp.float32),
            pltpu.VMEM((2 * _T, _H), jnp.float32),
            pltpu.VMEM((2, _H), jnp.float32),
            pltpu.SemaphoreType.DMA,
        ],
    )(tok, seg, token_table, segment_table, pos_weight)


def kernel(tokens, segments, token_table, segment_table, pos_weight):
    b, t = tokens.shape
    h = token_table.shape[1]
    tok = tokens.reshape(b * t).astype(jnp.int32)
    seg = segments.reshape(b * t).astype(jnp.int32)
    out = _emb(tok, seg, token_table, segment_table, pos_weight)
    return out.reshape(b, t, h)


# speedup vs baseline: 4.0606x; 1.1572x over previous
import functools

import jax
import jax.numpy as jnp
from jax import lax
from jax.experimental import pallas as pl
from jax.experimental.pallas import tpu as pltpu
from jax.experimental.pallas import tpu_sc as plsc

_H = 128
_T = 200
_NW = 32
_ROWS_PER_W = 32
_HV = _H // 16


def _emb_body(tok_hbm, seg_hbm, table_hbm, segtab_hbm, pos_hbm, out_hbm,
              tok_v, seg_v, rows_v, posab_v, segtab_v, sem):
    c = lax.axis_index("c")
    s = lax.axis_index("s")
    wid = s * 2 + c
    base0 = wid * (_ROWS_PER_W * _T)

    pltpu.sync_copy(pos_hbm, rows_v.at[0])
    pltpu.sync_copy(segtab_hbm, segtab_v)

    def pab_body(r, carry):
        for h in range(_HV):
            sl = pl.ds(h * 16, 16)
            p = rows_v[0, r, sl]
            posab_v[r, sl] = p + segtab_v[0, sl]
            posab_v[_T + r, sl] = p + segtab_v[1, sl]
        return carry

    lax.fori_loop(0, _T, pab_body, 0)

    def row_body(j, carry):
        base = base0 + j * _T
        pltpu.sync_copy(tok_hbm.at[pl.ds(base, 128)], tok_v.at[pl.ds(0, 128)])
        pltpu.sync_copy(tok_hbm.at[pl.ds(base + 128, _T - 128)],
                        tok_v.at[pl.ds(128, _T - 128)])
        pltpu.sync_copy(seg_hbm.at[pl.ds(base, _T)], seg_v.at[pl.ds(0, _T)])

        def comp(rv, carry2):
            sv = seg_v[pl.ds(rv * 16, 16)]
            av = sv * _T + (rv * 16 + lax.iota(jnp.int32, 16))
            for u in range(16):
                r = rv * 16 + u
                a = av[u]
                for h in range(_HV):
                    sl = pl.ds(h * 16, 16)
                    rows_v[0, r, sl] = posab_v[a, sl]
            return carry2

        lax.fori_loop(0, _T // 16, comp, 0)
        sv = seg_v[pl.ds(192, 16)]
        av = sv * _T + (192 + lax.iota(jnp.int32, 16))
        for u in range(_T - 192):
            r = 192 + u
            a = av[u]
            for h in range(_HV):
                sl = pl.ds(h * 16, 16)
                rows_v[0, r, sl] = posab_v[a, sl]

        cp1 = pltpu.async_copy(table_hbm.at[tok_v.at[pl.ds(0, 128)]],
                               rows_v.at[0, pl.ds(0, 128)], sem, add=True)
        cp2 = pltpu.async_copy(table_hbm.at[tok_v.at[pl.ds(128, _T - 128)]],
                               rows_v.at[0, pl.ds(128, _T - 128)], sem, add=True)
        cp1.wait()
        cp2.wait()
        pltpu.sync_copy(rows_v.at[0], out_hbm.at[pl.ds(base, _T)])
        return carry

    lax.fori_loop(0, _ROWS_PER_W, row_body, 0)


def _emb(tok, seg, token_table, segment_table, pos_weight):
    n = tok.shape[0]
    mesh = plsc.VectorSubcoreMesh(core_axis_name="c", subcore_axis_name="s")
    return pl.kernel(
        _emb_body,
        out_type=jax.ShapeDtypeStruct((n, _H), jnp.float32),
        mesh=mesh,
        scratch_types=[
            pltpu.VMEM((256,), jnp.int32),
            pltpu.VMEM((256,), jnp.int32),
            pltpu.VMEM((1, _T, _H), jnp.float32),
            pltpu.VMEM((2 * _T, _H), jnp.float32),
            pltpu.VMEM((2, _H), jnp.float32),
            pltpu.SemaphoreType.DMA,
        ],
    )(tok, seg, token_table, segment_table, pos_weight)


def kernel(tokens, segments, token_table, segment_table, pos_weight):
    b, t = tokens.shape
    h = token_table.shape[1]
    tok = tokens.reshape(b * t).astype(jnp.int32)
    seg = segments.reshape(b * t).astype(jnp.int32)
    out = _emb(tok, seg, token_table, segment_table, pos_weight)
    return out.reshape(b, t, h)


# speedup vs baseline: 11.5379x; 2.8415x over previous
import jax
import jax.numpy as jnp
from jax import lax
from jax.experimental import pallas as pl
from jax.experimental.pallas import tpu as pltpu
from jax.experimental.pallas import tpu_sc as plsc

_H = 128
_T = 200
_NW = 32
_ROWS_PER_W = 32
_HV = _H // 16
_NPW = _ROWS_PER_W * _T
_T1 = 128
_T2 = _T - _T1


def _emb_body(tok_hbm, seg_hbm, table_hbm, segtab_hbm, pos_hbm, out_hbm,
              tokall_v, segall_v, idx2_v, rows_v, segtab_v, posab_sp,
              asem, gsem, ssem):
    c = lax.axis_index("c")
    s = lax.axis_index("s")
    wid = s * 2 + c
    base0 = wid * _NPW

    pltpu.sync_copy(tok_hbm.at[pl.ds(base0, _NPW)], tokall_v.at[pl.ds(0, _NPW)])
    pltpu.sync_copy(seg_hbm.at[pl.ds(base0, _NPW)], segall_v.at[pl.ds(0, _NPW)])

    @pl.when(s == 0)
    def _build():
        pltpu.sync_copy(pos_hbm, rows_v.at[1])
        pltpu.sync_copy(segtab_hbm, segtab_v)
        for sidx in range(2):
            def pab_body(r, carry):
                for h in range(_HV):
                    sl = pl.ds(h * 16, 16)
                    rows_v[0, r, sl] = rows_v[1, r, sl] + segtab_v[sidx, sl]
                return carry
            lax.fori_loop(0, _T, pab_body, 0)
            pltpu.sync_copy(rows_v.at[0], posab_sp.at[pl.ds(sidx * _T, _T)])

    plsc.subcore_barrier()

    def _mk_idx2(j, b):
        for g in range(13):
            sv = segall_v[pl.ds(j * _T + g * 16, 16)]
            iv = sv * _T + (g * 16 + lax.iota(jnp.int32, 16))
            idx2_v[b, pl.ds(g * 16, 16)] = iv

    def _issue_a(j, b):
        pltpu.async_copy(posab_sp.at[idx2_v.at[b, pl.ds(0, _T1)]],
                         rows_v.at[b, pl.ds(0, _T1)], asem)
        pltpu.async_copy(posab_sp.at[idx2_v.at[b, pl.ds(_T1, _T2)]],
                         rows_v.at[b, pl.ds(_T1, _T2)], asem)

    def _issue_g(j, b):
        base = j * _T
        pltpu.async_copy(table_hbm.at[tokall_v.at[pl.ds(base, _T1)]],
                         rows_v.at[b, pl.ds(0, _T1)], gsem, add=True)
        pltpu.async_copy(table_hbm.at[tokall_v.at[pl.ds(base + _T1, _T2)]],
                         rows_v.at[b, pl.ds(_T1, _T2)], gsem, add=True)

    def _issue_s(j, b):
        pltpu.async_copy(rows_v.at[b], out_hbm.at[pl.ds(base0 + j * _T, _T)],
                         ssem)

    def _drain(sem, b):
        pltpu.make_async_copy(out_hbm.at[pl.ds(0, _T)], rows_v.at[b], sem).wait()

    def _drain_s(b):
        pltpu.make_async_copy(rows_v.at[b], out_hbm.at[pl.ds(0, _T)], ssem).wait()

    _mk_idx2(0, 0)
    _issue_a(0, 0)

    def outer(i, carry):
        for b in range(2):
            j = 2 * i + b
            _drain(asem, b)
            _issue_g(j, b)

            @pl.when(j >= 1)
            def _wait_prev_scatter():
                _drain_s(1 - b)

            @pl.when(j + 1 < _ROWS_PER_W)
            def _prime_next():
                _mk_idx2(j + 1, 1 - b)
                _issue_a(j + 1, 1 - b)

            _drain(gsem, b)
            _issue_s(j, b)
        return carry

    lax.fori_loop(0, _ROWS_PER_W // 2, outer, 0)
    _drain_s(1)


def _emb(tok, seg, token_table, segment_table, pos_weight):
    n = tok.shape[0]
    mesh = plsc.VectorSubcoreMesh(core_axis_name="c", subcore_axis_name="s")
    return pl.kernel(
        _emb_body,
        out_type=jax.ShapeDtypeStruct((n, _H), jnp.float32),
        mesh=mesh,
        scratch_types=[
            pltpu.VMEM((_NPW + 16,), jnp.int32),
            pltpu.VMEM((_NPW + 16,), jnp.int32),
            pltpu.VMEM((2, 208), jnp.int32),
            pltpu.VMEM((2, _T, _H), jnp.float32),
            pltpu.VMEM((2, _H), jnp.float32),
            pltpu.VMEM_SHARED((2 * _T, _H), jnp.float32),
            pltpu.SemaphoreType.DMA,
            pltpu.SemaphoreType.DMA,
            pltpu.SemaphoreType.DMA,
        ],
    )(tok, seg, token_table, segment_table, pos_weight)


def kernel(tokens, segments, token_table, segment_table, pos_weight):
    b, t = tokens.shape
    h = token_table.shape[1]
    tok = tokens.reshape(b * t).astype(jnp.int32)
    seg = segments.reshape(b * t).astype(jnp.int32)
    out = _emb(tok, seg, token_table, segment_table, pos_weight)
    return out.reshape(b, t, h)


# speedup vs baseline: 11.5600x; 1.0019x over previous
import jax
import jax.numpy as jnp
from jax import lax
from jax.experimental import pallas as pl
from jax.experimental.pallas import tpu as pltpu
from jax.experimental.pallas import tpu_sc as plsc

_H = 128
_T = 200
_NW = 32
_ROWS_PER_W = 32
_HV = _H // 16
_NPW = _ROWS_PER_W * _T
_T1 = 128
_T2 = _T - _T1
_BR = 25


def _emb_body(tok_hbm, seg_hbm, table_hbm, segtab_hbm, pos_hbm, out_hbm,
              tokall_v, segall_v, idx2a_v, idx2b_v, idx2c_v, rows_v,
              segtab_v, posab_sp, asem, gsem, ssem):
    idx2 = (idx2a_v, idx2b_v, idx2c_v)
    c = lax.axis_index("c")
    s = lax.axis_index("s")
    wid = s * 2 + c
    base0 = wid * _NPW

    pltpu.sync_copy(tok_hbm.at[pl.ds(base0, _NPW)], tokall_v.at[pl.ds(0, _NPW)])
    pltpu.sync_copy(seg_hbm.at[pl.ds(base0, _NPW)], segall_v.at[pl.ds(0, _NPW)])

    @pl.when(s == 0)
    def _build():
        pltpu.sync_copy(pos_hbm, rows_v.at[1])
        pltpu.sync_copy(segtab_hbm, segtab_v)
        for sidx in range(2):
            def pab_body(r, carry):
                for h in range(_HV):
                    sl = pl.ds(h * 16, 16)
                    rows_v[0, r, sl] = rows_v[1, r, sl] + segtab_v[sidx, sl]
                return carry
            lax.fori_loop(0, _T, pab_body, 0)
            pltpu.sync_copy(rows_v.at[0], posab_sp.at[pl.ds(sidx * _T, _T)])

    plsc.subcore_barrier()

    def _mk_idx2(j, b):
        for g in range(13):
            sv = segall_v[pl.ds(j * _T + g * 16, 16)]
            iv = sv * _T + (g * 16 + lax.iota(jnp.int32, 16))
            idx2[b][pl.ds(g * 16, 16)] = iv

    def _issue_a(j, b):
        pltpu.async_copy(posab_sp.at[idx2[b].at[pl.ds(0, _T1)]],
                         rows_v.at[b, pl.ds(0, _T1)], asem)
        pltpu.async_copy(posab_sp.at[idx2[b].at[pl.ds(_T1, _T2)]],
                         rows_v.at[b, pl.ds(_T1, _T2)], asem)

    def _issue_g(j, b):
        base = j * _T
        pltpu.async_copy(table_hbm.at[tokall_v.at[pl.ds(base, _T1)]],
                         rows_v.at[b, pl.ds(0, _T1)], gsem, add=True)
        pltpu.async_copy(table_hbm.at[tokall_v.at[pl.ds(base + _T1, _T2)]],
                         rows_v.at[b, pl.ds(_T1, _T2)], gsem, add=True)

    def _issue_s(j, b):
        pltpu.async_copy(rows_v.at[b], out_hbm.at[pl.ds(base0 + j * _T, _T)],
                         ssem)

    def _drain(sem, b):
        pltpu.make_async_copy(out_hbm.at[pl.ds(0, _T)], rows_v.at[b], sem).wait()

    def _drain_s(b):
        pltpu.make_async_copy(rows_v.at[b], out_hbm.at[pl.ds(0, _T)], ssem).wait()

    _mk_idx2(0, 0)
    _issue_a(0, 0)
    _mk_idx2(1, 1)
    _issue_a(1, 1)

    for j in range(_ROWS_PER_W):
        b = j % 3
        _drain(asem, b)
        _issue_g(j, b)
        if j >= 2:
            _drain_s((j + 1) % 3)
        if j + 2 < _ROWS_PER_W:
            _mk_idx2(j + 2, (j + 2) % 3)
            _issue_a(j + 2, (j + 2) % 3)
        _drain(gsem, b)
        _issue_s(j, b)
    _drain_s(30 % 3)
    _drain_s(31 % 3)


def _emb(tok, seg, token_table, segment_table, pos_weight):
    n = tok.shape[0]
    mesh = plsc.VectorSubcoreMesh(core_axis_name="c", subcore_axis_name="s")
    return pl.kernel(
        _emb_body,
        out_type=jax.ShapeDtypeStruct((n, _H), jnp.float32),
        mesh=mesh,
        scratch_types=[
            pltpu.VMEM((_NPW + 16,), jnp.int32),
            pltpu.VMEM((_NPW + 16,), jnp.int32),
            pltpu.VMEM((208,), jnp.int32),
            pltpu.VMEM((208,), jnp.int32),
            pltpu.VMEM((208,), jnp.int32),
            pltpu.VMEM((3, _T, _H), jnp.float32),
            pltpu.VMEM((2, _H), jnp.float32),
            pltpu.VMEM_SHARED((2 * _T, _H), jnp.float32),
            pltpu.SemaphoreType.DMA,
            pltpu.SemaphoreType.DMA,
            pltpu.SemaphoreType.DMA,
        ],
    )(tok, seg, token_table, segment_table, pos_weight)


def kernel(tokens, segments, token_table, segment_table, pos_weight):
    b, t = tokens.shape
    h = token_table.shape[1]
    tok = tokens.reshape(b * t).astype(jnp.int32)
    seg = segments.reshape(b * t).astype(jnp.int32)
    out = _emb(tok, seg, token_table, segment_table, pos_weight)
    return out.reshape(b, t, h)


# speedup vs baseline: 12.9658x; 1.1216x over previous
import jax
import jax.numpy as jnp
from jax import lax
from jax.experimental import pallas as pl
from jax.experimental.pallas import tpu as pltpu
from jax.experimental.pallas import tpu_sc as plsc

_H = 128
_T = 200
_NW = 32
_ROWS_PER_W = 32
_HV = _H // 16
_NPW = _ROWS_PER_W * _T
_T1 = 128
_T2 = _T - _T1
_BR = 25


def _emb_body(tok_hbm, seg_hbm, table_hbm, segtab_hbm, pos_hbm, out_hbm,
              tokall_v, segall_v, idx2a_v, idx2b_v, idx2c_v, idx2d_v, rows_v,
              segtab_v, posab_sp, asem, gsem, ssem):
    idx2 = (idx2a_v, idx2b_v, idx2c_v, idx2d_v)
    c = lax.axis_index("c")
    s = lax.axis_index("s")
    wid = s * 2 + c
    base0 = wid * _NPW

    pltpu.sync_copy(tok_hbm.at[pl.ds(base0, _NPW)], tokall_v.at[pl.ds(0, _NPW)])
    pltpu.sync_copy(seg_hbm.at[pl.ds(base0, _NPW)], segall_v.at[pl.ds(0, _NPW)])

    pltpu.sync_copy(segtab_hbm, segtab_v)

    def _build_block(blk):
        half = blk // _BR
        p0 = pl.multiple_of((blk % _BR) * 8, 8)
        hf = lax.broadcast_in_dim(half, (16,), ()).astype(jnp.float32)
        pltpu.sync_copy(pos_hbm.at[pl.ds(p0, 8)], rows_v.at[1, pl.ds(0, 8)])
        for r in range(8):
            for h in range(_HV):
                sl = pl.ds(h * 16, 16)
                s0 = segtab_v[0, sl]
                segv = s0 + hf * (segtab_v[1, sl] - s0)
                rows_v[0, r, sl] = rows_v[1, r, sl] + segv
        pltpu.sync_copy(rows_v.at[0, pl.ds(0, 8)],
                        posab_sp.at[pl.ds(pl.multiple_of(blk * 8, 8), 8)])

    for k in range(4):
        blk = s + 16 * k

        @pl.when(blk < 2 * _BR)
        def _do_build(blk=blk):
            _build_block(blk)

    plsc.subcore_barrier()

    def _mk_idx2(j, b):
        for g in range(13):
            sv = segall_v[pl.ds(j * _T + g * 16, 16)]
            iv = sv * _T + (g * 16 + lax.iota(jnp.int32, 16))
            idx2[b][pl.ds(g * 16, 16)] = iv

    def _issue_a(j, b):
        pltpu.async_copy(posab_sp.at[idx2[b].at[pl.ds(0, _T1)]],
                         rows_v.at[b, pl.ds(0, _T1)], asem)
        pltpu.async_copy(posab_sp.at[idx2[b].at[pl.ds(_T1, _T2)]],
                         rows_v.at[b, pl.ds(_T1, _T2)], asem)

    def _issue_g(j, b):
        base = j * _T
        pltpu.async_copy(table_hbm.at[tokall_v.at[pl.ds(base, _T1)]],
                         rows_v.at[b, pl.ds(0, _T1)], gsem, add=True)
        pltpu.async_copy(table_hbm.at[tokall_v.at[pl.ds(base + _T1, _T2)]],
                         rows_v.at[b, pl.ds(_T1, _T2)], gsem, add=True)

    def _issue_s(j, b):
        pltpu.async_copy(rows_v.at[b], out_hbm.at[pl.ds(base0 + j * _T, _T)],
                         ssem)

    def _drain(sem, b):
        pltpu.make_async_copy(out_hbm.at[pl.ds(0, _T)], rows_v.at[b], sem).wait()

    def _drain_s(b):
        pltpu.make_async_copy(rows_v.at[b], out_hbm.at[pl.ds(0, _T)], ssem).wait()

    _mk_idx2(0, 0)
    _issue_a(0, 0)
    _mk_idx2(1, 1)
    _issue_a(1, 1)

    for j in range(_ROWS_PER_W):
        b = j % 4
        _drain(asem, b)
        _issue_g(j, b)
        if j >= 2:
            _drain_s((j + 2) % 4)
        if j + 2 < _ROWS_PER_W:
            _mk_idx2(j + 2, (j + 2) % 4)
            _issue_a(j + 2, (j + 2) % 4)
        _drain(gsem, b)
        _issue_s(j, b)
    _drain_s(30 % 4)
    _drain_s(31 % 4)


def _emb(tok, seg, token_table, segment_table, pos_weight):
    n = tok.shape[0]
    mesh = plsc.VectorSubcoreMesh(core_axis_name="c", subcore_axis_name="s")
    return pl.kernel(
        _emb_body,
        out_type=jax.ShapeDtypeStruct((n, _H), jnp.float32),
        mesh=mesh,
        scratch_types=[
            pltpu.VMEM((_NPW + 16,), jnp.int32),
            pltpu.VMEM((_NPW + 16,), jnp.int32),
            pltpu.VMEM((208,), jnp.int32),
            pltpu.VMEM((208,), jnp.int32),
            pltpu.VMEM((208,), jnp.int32),
            pltpu.VMEM((208,), jnp.int32),
            pltpu.VMEM((4, _T, _H), jnp.float32),
            pltpu.VMEM((2, _H), jnp.float32),
            pltpu.VMEM_SHARED((2 * _T, _H), jnp.float32),
            pltpu.SemaphoreType.DMA,
            pltpu.SemaphoreType.DMA,
            pltpu.SemaphoreType.DMA,
        ],
    )(tok, seg, token_table, segment_table, pos_weight)


def kernel(tokens, segments, token_table, segment_table, pos_weight):
    b, t = tokens.shape
    h = token_table.shape[1]
    tok = tokens.reshape(b * t).astype(jnp.int32)
    seg = segments.reshape(b * t).astype(jnp.int32)
    out = _emb(tok, seg, token_table, segment_table, pos_weight)
    return out.reshape(b, t, h)
